# baseline (speedup 1.0000x reference)
import functools

import jax
import jax.numpy as jnp
from jax.experimental import pallas as pl
from jax.experimental.pallas import tpu as pltpu

_TV = 512
_TR = 1024


def _fused_mlp_kernel(x_ref, w1_ref, b1_ref, w2_ref, b2_ref, s_ref, o_ref,
                      xb_ref, *, n_v):
    v = pl.program_id(0)
    M = x_ref.shape[0]

    @pl.when(v == 0)
    def _():
        xb_ref[...] = x_ref[...].astype(jnp.bfloat16)

    w1c = w1_ref[...].astype(jnp.bfloat16)
    w2c = w2_ref[...].astype(jnp.bfloat16)
    b1c = b1_ref[...]
    for mt in range(M // _TR):
        rs = pl.ds(mt * _TR, _TR)
        h = jnp.dot(xb_ref[rs, :], w1c, preferred_element_type=jnp.float32)
        h = jnp.maximum(h + b1c, 0.0).astype(jnp.bfloat16)
        p = jnp.dot(h, w2c, preferred_element_type=jnp.float32)

        @pl.when(v == 0)
        def _():
            o_ref[rs, :] = p

        @pl.when(v > 0)
        def _():
            o_ref[rs, :] += p

    @pl.when(v == n_v - 1)
    def _():
        for mt in range(M // _TR):
            rs = pl.ds(mt * _TR, _TR)
            y = o_ref[rs, :] + b2_ref[...] + x_ref[rs, :]
            var = jnp.mean(y * y, axis=-1, keepdims=True)
            o_ref[rs, :] = (y * jax.lax.rsqrt(var + 1e-6)) * s_ref[...]


def kernel(x, W1, b1, W2, b2, ln_scale):
    B, S, D = x.shape
    K, V = W1.shape
    M = B * S
    n_v = V // _TV

    xf = x.reshape(M, D)
    b1r = b1.reshape(1, V)
    b2r = b2.reshape(1, D)
    snr = ln_scale.reshape(1, D)

    body = functools.partial(_fused_mlp_kernel, n_v=n_v)

    out = pl.pallas_call(
        body,
        grid=(n_v,),
        in_specs=[
            pl.BlockSpec((M, D), lambda v: (0, 0)),
            pl.BlockSpec((K, _TV), lambda v: (0, v)),
            pl.BlockSpec((1, _TV), lambda v: (0, v)),
            pl.BlockSpec((_TV, D), lambda v: (v, 0)),
            pl.BlockSpec((1, D), lambda v: (0, 0)),
            pl.BlockSpec((1, D), lambda v: (0, 0)),
        ],
        out_specs=pl.BlockSpec((M, D), lambda v: (0, 0)),
        out_shape=jax.ShapeDtypeStruct((M, D), jnp.float32),
        scratch_shapes=[pltpu.VMEM((M, D), jnp.bfloat16)],
        compiler_params=pltpu.CompilerParams(
            dimension_semantics=("arbitrary",),
        ),
    )(xf, W1, b1r, W2, b2r, snr)
    return out.reshape(B, S, D)


# speedup vs baseline: 1.2207x; 1.2207x over previous
import functools

import jax
import jax.numpy as jnp
from jax.experimental import pallas as pl
from jax.experimental.pallas import tpu as pltpu

_TM = 512
_C1 = 256
_C2 = 1024


def _fused_mlp_kernel(x_ref, w1_hbm, b1_ref, w2_hbm, b2_ref, s_ref, o_ref,
                      w1b_ref, w2b_ref, st1_ref, st2_ref, sem1, sem2, *, n_m):
    m = pl.program_id(0)

    @pl.when(m == 0)
    def _():
        n1 = w1_hbm.shape[0] // _C1
        n2 = w2_hbm.shape[0] // _C2

        def cp1(i):
            return pltpu.make_async_copy(
                w1_hbm.at[pl.ds(i * _C1, _C1), :],
                st1_ref.at[i % 2], sem1.at[i % 2])

        def cp2(i):
            return pltpu.make_async_copy(
                w2_hbm.at[pl.ds(i * _C2, _C2), :],
                st2_ref.at[i % 2], sem2.at[i % 2])

        cp1(0).start()
        cp1(1).start()
        for i in range(n1):
            cp1(i).wait()
            w1b_ref[pl.ds(i * _C1, _C1), :] = st1_ref[i % 2].astype(
                jnp.bfloat16)
            if i + 2 < n1:
                cp1(i + 2).start()
            elif i + 2 - n1 < 2:
                cp2(i + 2 - n1).start()
        for i in range(n2):
            cp2(i).wait()
            w2b_ref[pl.ds(i * _C2, _C2), :] = st2_ref[i % 2].astype(
                jnp.bfloat16)
            if i + 2 < n2:
                cp2(i + 2).start()

    @pl.when(m > 0)
    def _():
        x = x_ref[...]
        h = jnp.dot(x.astype(jnp.bfloat16), w1b_ref[...],
                    preferred_element_type=jnp.float32)
        h = jnp.maximum(h + b1_ref[...], 0.0).astype(jnp.bfloat16)
        y = jnp.dot(h, w2b_ref[...], preferred_element_type=jnp.float32)
        y = y + b2_ref[...] + x
        var = jnp.mean(y * y, axis=-1, keepdims=True)
        o_ref[...] = (y * jax.lax.rsqrt(var + 1e-6)) * s_ref[...]


def kernel(x, W1, b1, W2, b2, ln_scale):
    B, S, D = x.shape
    K, V = W1.shape
    M = B * S
    n_m = M // _TM

    xf = x.reshape(M, D)
    b1r = b1.reshape(1, V)
    b2r = b2.reshape(1, D)
    snr = ln_scale.reshape(1, D)

    body = functools.partial(_fused_mlp_kernel, n_m=n_m)

    out = pl.pallas_call(
        body,
        grid=(n_m + 1,),
        in_specs=[
            pl.BlockSpec((_TM, D), lambda m: (jnp.maximum(m - 1, 0), 0)),
            pl.BlockSpec(memory_space=pltpu.MemorySpace.HBM),
            pl.BlockSpec((1, V), lambda m: (0, 0)),
            pl.BlockSpec(memory_space=pltpu.MemorySpace.HBM),
            pl.BlockSpec((1, D), lambda m: (0, 0)),
            pl.BlockSpec((1, D), lambda m: (0, 0)),
        ],
        out_specs=pl.BlockSpec((_TM, D), lambda m: (jnp.maximum(m - 1, 0), 0)),
        out_shape=jax.ShapeDtypeStruct((M, D), jnp.float32),
        scratch_shapes=[
            pltpu.VMEM((K, V), jnp.bfloat16),
            pltpu.VMEM((V, D), jnp.bfloat16),
            pltpu.VMEM((2, _C1, V), jnp.float32),
            pltpu.VMEM((2, _C2, D), jnp.float32),
            pltpu.SemaphoreType.DMA((2,)),
            pltpu.SemaphoreType.DMA((2,)),
        ],
        compiler_params=pltpu.CompilerParams(
            dimension_semantics=("arbitrary",),
        ),
    )(xf, W1, b1r, W2, b2r, snr)
    return out.reshape(B, S, D)


# speedup vs baseline: 1.3069x; 1.0706x over previous
import jax
import jax.numpy as jnp
from jax.experimental import pallas as pl
from jax.experimental.pallas import tpu as pltpu

_TM = 512
_TC = 512


def _fused_mlp_kernel(x_ref, w1_hbm, b1_ref, w2_hbm, b2_ref, s_ref, o_ref,
                      w1b_ref, w2b_ref, st1_ref, st2_ref, sem1, sem2):
    m = pl.program_id(0)
    V = w1b_ref.shape[1]
    n_c = V // _TC

    @pl.when(m == 0)
    def _():
        def cp1(c):
            return pltpu.make_async_copy(
                w1_hbm.at[:, pl.ds(c * _TC, _TC)],
                st1_ref.at[c % 2], sem1.at[c % 2])

        def cp2(c):
            return pltpu.make_async_copy(
                w2_hbm.at[pl.ds(c * _TC, _TC), :],
                st2_ref.at[c % 2], sem2.at[c % 2])

        cp1(0).start()
        cp2(0).start()
        cp1(1).start()
        cp2(1).start()
        x = x_ref[...]
        xb = x.astype(jnp.bfloat16)
        acc = x + b2_ref[...]
        for c in range(n_c):
            sl = pl.ds(c * _TC, _TC)
            cp1(c).wait()
            w1b_ref[:, sl] = st1_ref[c % 2].astype(jnp.bfloat16)
            if c + 2 < n_c:
                cp1(c + 2).start()
            h = jnp.dot(xb, w1b_ref[:, sl],
                        preferred_element_type=jnp.float32)
            h = jnp.maximum(h + b1_ref[:, sl], 0.0).astype(jnp.bfloat16)
            cp2(c).wait()
            w2b_ref[sl, :] = st2_ref[c % 2].astype(jnp.bfloat16)
            if c + 2 < n_c:
                cp2(c + 2).start()
            acc = acc + jnp.dot(h, w2b_ref[sl, :],
                                preferred_element_type=jnp.float32)
        var = jnp.mean(acc * acc, axis=-1, keepdims=True)
        o_ref[...] = (acc * jax.lax.rsqrt(var + 1e-6)) * s_ref[...]

    @pl.when(m > 0)
    def _():
        x = x_ref[...]
        h = jnp.dot(x.astype(jnp.bfloat16), w1b_ref[...],
                    preferred_element_type=jnp.float32)
        h = jnp.maximum(h + b1_ref[...], 0.0).astype(jnp.bfloat16)
        y = jnp.dot(h, w2b_ref[...], preferred_element_type=jnp.float32)
        y = y + b2_ref[...] + x
        var = jnp.mean(y * y, axis=-1, keepdims=True)
        o_ref[...] = (y * jax.lax.rsqrt(var + 1e-6)) * s_ref[...]


def kernel(x, W1, b1, W2, b2, ln_scale):
    B, S, D = x.shape
    K, V = W1.shape
    M = B * S
    n_m = M // _TM

    xf = x.reshape(M, D)
    b1r = b1.reshape(1, V)
    b2r = b2.reshape(1, D)
    snr = ln_scale.reshape(1, D)

    out = pl.pallas_call(
        _fused_mlp_kernel,
        grid=(n_m,),
        in_specs=[
            pl.BlockSpec((_TM, D), lambda m: (m, 0)),
            pl.BlockSpec(memory_space=pltpu.MemorySpace.HBM),
            pl.BlockSpec((1, V), lambda m: (0, 0)),
            pl.BlockSpec(memory_space=pltpu.MemorySpace.HBM),
            pl.BlockSpec((1, D), lambda m: (0, 0)),
            pl.BlockSpec((1, D), lambda m: (0, 0)),
        ],
        out_specs=pl.BlockSpec((_TM, D), lambda m: (m, 0)),
        out_shape=jax.ShapeDtypeStruct((M, D), jnp.float32),
        scratch_shapes=[
            pltpu.VMEM((K, V), jnp.bfloat16),
            pltpu.VMEM((V, D), jnp.bfloat16),
            pltpu.VMEM((2, K, _TC), jnp.float32),
            pltpu.VMEM((2, _TC, D), jnp.float32),
            pltpu.SemaphoreType.DMA((2,)),
            pltpu.SemaphoreType.DMA((2,)),
        ],
        compiler_params=pltpu.CompilerParams(
            dimension_semantics=("arbitrary",),
        ),
    )(xf, W1, b1r, W2, b2r, snr)
    return out.reshape(B, S, D)
